# baseline (speedup 1.0000x reference)
import functools

import jax
import jax.numpy as jnp
from jax import lax
from jax.experimental import pallas as pl
from jax.experimental.pallas import tpu as pltpu
from jax.experimental.pallas import tpu_sc as plsc

N = 10000
IN_DIM = 256
HID_DIM = 512
OUT_DIM = 256
E = 160000

NPAD = 10240
EPAD = 163840
BLK = 128
IBLK = 128
NSUB = 16
NCORE = 2
RPT = NPAD // NSUB
EPT = EPAD // NSUB
NBLK = EPT // IBLK
HBLK = EPAD // (NSUB * NCORE * IBLK)
CH = 8
NCH = NBLK // CH

_MESH = plsc.VectorSubcoreMesh(core_axis_name="c", subcore_axis_name="s")


def _sc_hist(dst2d, zeros_col, ones_col):
    @functools.partial(
        pl.kernel,
        out_type=jax.ShapeDtypeStruct((NCORE, NPAD, BLK), jnp.float32),
        mesh=_MESH,
        scratch_types=[
            pltpu.VMEM_SHARED((NPAD, BLK), jnp.float32),
            pltpu.VMEM((HBLK, IBLK), jnp.int32),
            pltpu.VMEM((IBLK, BLK), jnp.float32),
        ],
    )
    def hist(dst_hbm, zeros_hbm, ones_hbm, deg_hbm, acc, dst_v, ones_v):
        c = lax.axis_index("c")
        w = lax.axis_index("s")
        r0 = w * RPT
        pltpu.sync_copy(zeros_hbm.at[pl.ds(r0, RPT)], acc.at[pl.ds(r0, RPT)])
        pltpu.sync_copy(ones_hbm, ones_v)
        blk0 = (c * NSUB + w) * HBLK
        pltpu.sync_copy(dst_hbm.at[pl.ds(blk0, HBLK)], dst_v)
        plsc.subcore_barrier()

        @pl.loop(0, HBLK)
        def _(j):
            pltpu.sync_copy(ones_v, acc.at[dst_v.at[j]], add=True)

        plsc.subcore_barrier()
        pltpu.sync_copy(acc.at[pl.ds(r0, RPT)], deg_hbm.at[c, pl.ds(r0, RPT)])

    return hist(dst2d, zeros_col, ones_col)


def _sc_prop(y_flat, src_sh, dst2d, n_slices):
    passes = n_slices // NCORE
    nbr = EPAD // IBLK

    @functools.partial(
        pl.kernel,
        out_type=jax.ShapeDtypeStruct((n_slices * NPAD, BLK), jnp.float32),
        mesh=_MESH,
        scratch_types=[
            pltpu.VMEM_SHARED((NPAD, BLK), jnp.float32),
            pltpu.VMEM((2, CH, IBLK), jnp.int32),
            pltpu.VMEM((2, CH, IBLK), jnp.int32),
            pltpu.VMEM((2, IBLK, BLK), jnp.float32),
            pltpu.SemaphoreType.DMA,
            pltpu.SemaphoreType.DMA,
        ],
    )
    def prop(y_hbm, src_hbm, dst_hbm, agg_hbm, acc, src_c, dst_c, buf, gsem, isem):
        c = lax.axis_index("c")
        w = lax.axis_index("s")
        r0 = w * RPT

        for p in range(passes):
            sl = c * passes + p
            srow0 = sl * nbr + w * NBLK
            drow0 = w * NBLK
            pltpu.sync_copy(
                y_hbm.at[pl.ds(sl * NPAD + r0, RPT)], acc.at[pl.ds(r0, RPT)]
            )
            pltpu.make_async_copy(
                src_hbm.at[pl.ds(srow0, CH)], src_c.at[0], isem
            ).start()
            pltpu.make_async_copy(
                dst_hbm.at[pl.ds(drow0, CH)], dst_c.at[0], isem
            ).start()
            plsc.subcore_barrier()

            @pl.loop(0, NCH, step=2)
            def _(kk):
                for cb in range(2):
                    k = kk + cb
                    pltpu.make_async_copy(
                        src_hbm.at[pl.ds(0, CH)], src_c.at[cb], isem
                    ).wait()
                    pltpu.make_async_copy(
                        src_hbm.at[pl.ds(0, CH)], dst_c.at[cb], isem
                    ).wait()

                    @pl.when(k < NCH - 1)
                    def _():
                        pltpu.make_async_copy(
                            src_hbm.at[pl.ds(srow0 + (k + 1) * CH, CH)],
                            src_c.at[1 - cb], isem,
                        ).start()
                        pltpu.make_async_copy(
                            dst_hbm.at[pl.ds(drow0 + (k + 1) * CH, CH)],
                            dst_c.at[1 - cb], isem,
                        ).start()

                    pltpu.make_async_copy(
                        y_hbm.at[src_c.at[cb, 0]], buf.at[0], gsem
                    ).start()
                    for j2 in range(CH):
                        bb = j2 % 2
                        pltpu.make_async_copy(
                            y_hbm.at[pl.ds(0, IBLK)], buf.at[bb], gsem
                        ).wait()
                        if j2 < CH - 1:
                            pltpu.make_async_copy(
                                y_hbm.at[src_c.at[cb, j2 + 1]],
                                buf.at[1 - bb], gsem,
                            ).start()
                        pltpu.sync_copy(
                            buf.at[bb], acc.at[dst_c.at[cb, j2]], add=True
                        )

            plsc.subcore_barrier()
            pltpu.sync_copy(
                acc.at[pl.ds(r0, RPT)], agg_hbm.at[pl.ds(sl * NPAD + r0, RPT)]
            )
            if p < passes - 1:
                plsc.subcore_barrier()

    return prop(y_flat, src_sh, dst2d)


def _dinv_of(deg_ref):
    return lax.rsqrt(deg_ref[0][:, :1] + deg_ref[1][:, :1] + 1.0)


def _mm1_body(x_ref, w_ref, deg_ref, out_ref):
    d = _dinv_of(deg_ref)
    y = jnp.dot(x_ref[...] * d, w_ref[...], preferred_element_type=jnp.float32)
    for s in range(HID_DIM // BLK):
        out_ref[s] = y[:, s * BLK:(s + 1) * BLK]


def _mm2_body(agg_ref, deg_ref, b_ref, w_ref, out_ref):
    d = _dinv_of(deg_ref)
    a = jnp.concatenate([agg_ref[s] for s in range(HID_DIM // BLK)], axis=1)
    h = jnp.maximum(a * d + b_ref[...], 0.0)
    y = jnp.dot(h * d, w_ref[...], preferred_element_type=jnp.float32)
    for s in range(OUT_DIM // BLK):
        out_ref[s] = y[:, s * BLK:(s + 1) * BLK]


def _fin_body(agg_ref, deg_ref, b_ref, out_ref):
    d = _dinv_of(deg_ref)
    a = jnp.concatenate([agg_ref[s] for s in range(OUT_DIM // BLK)], axis=1)
    out_ref[...] = a * d + b_ref[...]


_RB = 256
_GRID = (NPAD // _RB,)


def _deg_spec():
    return pl.BlockSpec((NCORE, _RB, BLK), lambda i: (0, i, 0))


def kernel(x, edge_index, W1, b1, W2, b2):
    src = edge_index[0].astype(jnp.int32)
    dst = edge_index[1].astype(jnp.int32)
    pad_e = EPAD - E
    src2d = jnp.concatenate([src, jnp.zeros((pad_e,), jnp.int32)]).reshape(
        EPAD // IBLK, IBLK
    )
    dummy_dst = N + (jnp.arange(pad_e, dtype=jnp.int32) % (NPAD - N))
    dst2d = jnp.concatenate([dst, dummy_dst]).reshape(EPAD // IBLK, IBLK)
    xp = jnp.concatenate([x, jnp.zeros((NPAD - N, IN_DIM), x.dtype)])
    zeros_col = jnp.zeros((NPAD, BLK), jnp.float32)
    ones_col = jnp.ones((IBLK, BLK), jnp.float32)

    shifts = (jnp.arange(HID_DIM // BLK, dtype=jnp.int32) * NPAD)[:, None, None]
    src_sh = (src2d[None] + shifts).reshape((HID_DIM // BLK) * (EPAD // IBLK), IBLK)

    deg = _sc_hist(dst2d, zeros_col, ones_col)

    y1 = pl.pallas_call(
        _mm1_body,
        grid=_GRID,
        in_specs=[
            pl.BlockSpec((_RB, IN_DIM), lambda i: (i, 0)),
            pl.BlockSpec((IN_DIM, HID_DIM), lambda i: (0, 0)),
            _deg_spec(),
        ],
        out_specs=pl.BlockSpec((HID_DIM // BLK, _RB, BLK), lambda i: (0, i, 0)),
        out_shape=jax.ShapeDtypeStruct((HID_DIM // BLK, NPAD, BLK), jnp.float32),
    )(xp, W1, deg)

    agg1 = _sc_prop(
        y1.reshape((HID_DIM // BLK) * NPAD, BLK), src_sh, dst2d, HID_DIM // BLK
    ).reshape(HID_DIM // BLK, NPAD, BLK)

    y2 = pl.pallas_call(
        _mm2_body,
        grid=_GRID,
        in_specs=[
            pl.BlockSpec((HID_DIM // BLK, _RB, BLK), lambda i: (0, i, 0)),
            _deg_spec(),
            pl.BlockSpec((1, HID_DIM), lambda i: (0, 0)),
            pl.BlockSpec((HID_DIM, OUT_DIM), lambda i: (0, 0)),
        ],
        out_specs=pl.BlockSpec((OUT_DIM // BLK, _RB, BLK), lambda i: (0, i, 0)),
        out_shape=jax.ShapeDtypeStruct((OUT_DIM // BLK, NPAD, BLK), jnp.float32),
    )(agg1, deg, b1.reshape(1, HID_DIM), W2)

    agg2 = _sc_prop(
        y2.reshape((OUT_DIM // BLK) * NPAD, BLK), src_sh, dst2d, OUT_DIM // BLK
    ).reshape(OUT_DIM // BLK, NPAD, BLK)

    out = pl.pallas_call(
        _fin_body,
        grid=_GRID,
        in_specs=[
            pl.BlockSpec((OUT_DIM // BLK, _RB, BLK), lambda i: (0, i, 0)),
            _deg_spec(),
            pl.BlockSpec((1, OUT_DIM), lambda i: (0, 0)),
        ],
        out_specs=pl.BlockSpec((_RB, OUT_DIM), lambda i: (i, 0)),
        out_shape=jax.ShapeDtypeStruct((N, OUT_DIM), jnp.float32),
    )(agg2, deg, b2.reshape(1, OUT_DIM))

    return out


# speedup vs baseline: 2.0256x; 2.0256x over previous
import functools

import jax
import jax.numpy as jnp
from jax import lax
from jax.experimental import pallas as pl
from jax.experimental.pallas import tpu as pltpu
from jax.experimental.pallas import tpu_sc as plsc

N = 10000
IN_DIM = 256
HID_DIM = 512
OUT_DIM = 256
E = 160000

NPAD = 10240
EPAD = 163840
BLK = 128
IBLK = 128
NSUB = 16
NCORE = 2
RPT = NPAD // NSUB
EPT = EPAD // NSUB
NBLK = EPT // IBLK
HBLK = EPAD // (NSUB * NCORE * IBLK)
CH = 8
NCH = NBLK // CH

_MESH = plsc.VectorSubcoreMesh(core_axis_name="c", subcore_axis_name="s")


def _sc_hist(dst2d, zeros_col, ones_col):
    @functools.partial(
        pl.kernel,
        out_type=jax.ShapeDtypeStruct((NCORE, NPAD, BLK), jnp.float32),
        mesh=_MESH,
        scratch_types=[
            pltpu.VMEM_SHARED((NPAD, BLK), jnp.float32),
            pltpu.VMEM((HBLK, IBLK), jnp.int32),
            pltpu.VMEM((IBLK, BLK), jnp.float32),
        ],
    )
    def hist(dst_hbm, zeros_hbm, ones_hbm, deg_hbm, acc, dst_v, ones_v):
        c = lax.axis_index("c")
        w = lax.axis_index("s")
        r0 = w * RPT
        pltpu.sync_copy(zeros_hbm.at[pl.ds(r0, RPT)], acc.at[pl.ds(r0, RPT)])
        pltpu.sync_copy(ones_hbm, ones_v)
        blk0 = (c * NSUB + w) * HBLK
        pltpu.sync_copy(dst_hbm.at[pl.ds(blk0, HBLK)], dst_v)
        plsc.subcore_barrier()

        @pl.loop(0, HBLK)
        def _(j):
            pltpu.sync_copy(ones_v, acc.at[dst_v.at[j]], add=True)

        plsc.subcore_barrier()
        pltpu.sync_copy(acc.at[pl.ds(r0, RPT)], deg_hbm.at[c, pl.ds(r0, RPT)])

    return hist(dst2d, zeros_col, ones_col)


def _sc_prop(y_flat, src_sh, dst2d, n_slices):
    passes = n_slices // NCORE
    nbr = EPAD // IBLK

    @functools.partial(
        pl.kernel,
        out_type=jax.ShapeDtypeStruct((n_slices * NPAD, BLK), jnp.float32),
        mesh=_MESH,
        scratch_types=[
            pltpu.VMEM_SHARED((NPAD, BLK), jnp.float32),
            pltpu.VMEM((2, CH, IBLK), jnp.int32),
            pltpu.VMEM((2, CH, IBLK), jnp.int32),
            pltpu.VMEM((2, IBLK, BLK), jnp.float32),
            pltpu.SemaphoreType.DMA,
            pltpu.SemaphoreType.DMA,
        ],
    )
    def prop(y_hbm, src_hbm, dst_hbm, agg_hbm, acc, src_c, dst_c, buf, gsem, isem):
        c = lax.axis_index("c")
        w = lax.axis_index("s")
        r0 = w * RPT

        for p in range(passes):
            sl = c * passes + p
            srow0 = sl * nbr + w * NBLK
            drow0 = w * NBLK
            pltpu.sync_copy(
                y_hbm.at[pl.ds(sl * NPAD + r0, RPT)], acc.at[pl.ds(r0, RPT)]
            )
            pltpu.make_async_copy(
                src_hbm.at[pl.ds(srow0, CH)], src_c.at[0], isem
            ).start()
            pltpu.make_async_copy(
                dst_hbm.at[pl.ds(drow0, CH)], dst_c.at[0], isem
            ).start()
            plsc.subcore_barrier()

            @pl.loop(0, NCH, step=2)
            def _(kk):
                for cb in range(2):
                    k = kk + cb
                    pltpu.make_async_copy(
                        src_hbm.at[pl.ds(0, CH)], src_c.at[cb], isem
                    ).wait()
                    pltpu.make_async_copy(
                        src_hbm.at[pl.ds(0, CH)], dst_c.at[cb], isem
                    ).wait()

                    @pl.when(k < NCH - 1)
                    def _():
                        pltpu.make_async_copy(
                            src_hbm.at[pl.ds(srow0 + (k + 1) * CH, CH)],
                            src_c.at[1 - cb], isem,
                        ).start()
                        pltpu.make_async_copy(
                            dst_hbm.at[pl.ds(drow0 + (k + 1) * CH, CH)],
                            dst_c.at[1 - cb], isem,
                        ).start()

                    pltpu.make_async_copy(
                        y_hbm.at[src_c.at[cb, 0]], buf.at[0], gsem
                    ).start()
                    for j2 in range(CH):
                        bb = j2 % 2
                        pltpu.make_async_copy(
                            y_hbm.at[pl.ds(0, IBLK)], buf.at[bb], gsem
                        ).wait()
                        if j2 < CH - 1:
                            pltpu.make_async_copy(
                                y_hbm.at[src_c.at[cb, j2 + 1]],
                                buf.at[1 - bb], gsem,
                            ).start()
                        pltpu.sync_copy(
                            buf.at[bb], acc.at[dst_c.at[cb, j2]], add=True
                        )

            plsc.subcore_barrier()
            pltpu.sync_copy(
                acc.at[pl.ds(r0, RPT)], agg_hbm.at[pl.ds(sl * NPAD + r0, RPT)]
            )
            if p < passes - 1:
                plsc.subcore_barrier()

    return prop(y_flat, src_sh, dst2d)


def _dinv_of(deg_ref):
    return lax.rsqrt(deg_ref[0][:, :1] + deg_ref[1][:, :1] + 1.0)


def _mm1_body(x_ref, w_ref, deg_ref, out_ref):
    d = _dinv_of(deg_ref)
    y = jnp.dot(x_ref[...] * d, w_ref[...], preferred_element_type=jnp.float32)
    for s in range(HID_DIM // BLK):
        out_ref[s] = y[:, s * BLK:(s + 1) * BLK]


def _mm2_body(agg_ref, deg_ref, b_ref, w_ref, out_ref):
    d = _dinv_of(deg_ref)
    a = jnp.concatenate([agg_ref[s] for s in range(HID_DIM // BLK)], axis=1)
    h = jnp.maximum(a * d + b_ref[...], 0.0)
    y = jnp.dot(h * d, w_ref[...], preferred_element_type=jnp.float32)
    for s in range(OUT_DIM // BLK):
        out_ref[s] = y[:, s * BLK:(s + 1) * BLK]


def _fin_body(agg_ref, deg_ref, b_ref, out_ref):
    d = _dinv_of(deg_ref)
    a = jnp.concatenate([agg_ref[s] for s in range(OUT_DIM // BLK)], axis=1)
    out_ref[...] = a * d + b_ref[...]


_RB = 256
_GRID = (NPAD // _RB,)


def _deg_spec():
    return pl.BlockSpec((NCORE, _RB, BLK), lambda i: (0, i, 0))


def kernel(x, edge_index, W1, b1, W2, b2):
    src = edge_index[0].astype(jnp.int32)
    dst = edge_index[1].astype(jnp.int32)
    pad_e = EPAD - E
    dummy_src = jnp.arange(pad_e, dtype=jnp.int32) % N
    src2d = jnp.concatenate([src, dummy_src]).reshape(EPAD // IBLK, IBLK)
    dummy_dst = N + (jnp.arange(pad_e, dtype=jnp.int32) % (NPAD - N))
    dst2d = jnp.concatenate([dst, dummy_dst]).reshape(EPAD // IBLK, IBLK)
    xp = jnp.concatenate([x, jnp.zeros((NPAD - N, IN_DIM), x.dtype)])
    zeros_col = jnp.zeros((NPAD, BLK), jnp.float32)
    ones_col = jnp.ones((IBLK, BLK), jnp.float32)

    shifts = (jnp.arange(HID_DIM // BLK, dtype=jnp.int32) * NPAD)[:, None, None]
    src_sh = (src2d[None] + shifts).reshape((HID_DIM // BLK) * (EPAD // IBLK), IBLK)

    deg = _sc_hist(dst2d, zeros_col, ones_col)

    y1 = pl.pallas_call(
        _mm1_body,
        grid=_GRID,
        in_specs=[
            pl.BlockSpec((_RB, IN_DIM), lambda i: (i, 0)),
            pl.BlockSpec((IN_DIM, HID_DIM), lambda i: (0, 0)),
            _deg_spec(),
        ],
        out_specs=pl.BlockSpec((HID_DIM // BLK, _RB, BLK), lambda i: (0, i, 0)),
        out_shape=jax.ShapeDtypeStruct((HID_DIM // BLK, NPAD, BLK), jnp.float32),
    )(xp, W1, deg)

    agg1 = _sc_prop(
        y1.reshape((HID_DIM // BLK) * NPAD, BLK), src_sh, dst2d, HID_DIM // BLK
    ).reshape(HID_DIM // BLK, NPAD, BLK)

    y2 = pl.pallas_call(
        _mm2_body,
        grid=_GRID,
        in_specs=[
            pl.BlockSpec((HID_DIM // BLK, _RB, BLK), lambda i: (0, i, 0)),
            _deg_spec(),
            pl.BlockSpec((1, HID_DIM), lambda i: (0, 0)),
            pl.BlockSpec((HID_DIM, OUT_DIM), lambda i: (0, 0)),
        ],
        out_specs=pl.BlockSpec((OUT_DIM // BLK, _RB, BLK), lambda i: (0, i, 0)),
        out_shape=jax.ShapeDtypeStruct((OUT_DIM // BLK, NPAD, BLK), jnp.float32),
    )(agg1, deg, b1.reshape(1, HID_DIM), W2)

    agg2 = _sc_prop(
        y2.reshape((OUT_DIM // BLK) * NPAD, BLK), src_sh, dst2d, OUT_DIM // BLK
    ).reshape(OUT_DIM // BLK, NPAD, BLK)

    out = pl.pallas_call(
        _fin_body,
        grid=_GRID,
        in_specs=[
            pl.BlockSpec((OUT_DIM // BLK, _RB, BLK), lambda i: (0, i, 0)),
            _deg_spec(),
            pl.BlockSpec((1, OUT_DIM), lambda i: (0, 0)),
        ],
        out_specs=pl.BlockSpec((_RB, OUT_DIM), lambda i: (i, 0)),
        out_shape=jax.ShapeDtypeStruct((N, OUT_DIM), jnp.float32),
    )(agg2, deg, b2.reshape(1, OUT_DIM))

    return out


# speedup vs baseline: 2.0307x; 1.0025x over previous
import functools

import jax
import jax.numpy as jnp
from jax import lax
from jax.experimental import pallas as pl
from jax.experimental.pallas import tpu as pltpu
from jax.experimental.pallas import tpu_sc as plsc

N = 10000
IN_DIM = 256
HID_DIM = 512
OUT_DIM = 256
E = 160000

NPAD = 10240
EPAD = 163840
BLK = 128
IBLK = 128
NSUB = 16
NCORE = 2
RPT = NPAD // NSUB
EPT = EPAD // NSUB
NBLK = EPT // IBLK
HBLK = EPAD // (NSUB * NCORE * IBLK)
CH = 8
NCH = NBLK // CH

_MESH = plsc.VectorSubcoreMesh(core_axis_name="c", subcore_axis_name="s")


def _sc_hist(dst2d, zeros_col, ones_col):
    @functools.partial(
        pl.kernel,
        out_type=jax.ShapeDtypeStruct((NCORE, NPAD, BLK), jnp.float32),
        mesh=_MESH,
        scratch_types=[
            pltpu.VMEM_SHARED((NPAD, BLK), jnp.float32),
            pltpu.VMEM((HBLK, IBLK), jnp.int32),
            pltpu.VMEM((IBLK, BLK), jnp.float32),
        ],
    )
    def hist(dst_hbm, zeros_hbm, ones_hbm, deg_hbm, acc, dst_v, ones_v):
        c = lax.axis_index("c")
        w = lax.axis_index("s")
        r0 = w * RPT
        pltpu.sync_copy(zeros_hbm.at[pl.ds(r0, RPT)], acc.at[pl.ds(r0, RPT)])
        pltpu.sync_copy(ones_hbm, ones_v)
        blk0 = (c * NSUB + w) * HBLK
        pltpu.sync_copy(dst_hbm.at[pl.ds(blk0, HBLK)], dst_v)
        plsc.subcore_barrier()

        @pl.loop(0, HBLK)
        def _(j):
            pltpu.sync_copy(ones_v, acc.at[dst_v.at[j]], add=True)

        plsc.subcore_barrier()
        pltpu.sync_copy(acc.at[pl.ds(r0, RPT)], deg_hbm.at[c, pl.ds(r0, RPT)])

    return hist(dst2d, zeros_col, ones_col)


def _sc_prop(y_flat, src_sh, dst2d, n_slices):
    passes = n_slices // NCORE
    nbr = EPAD // IBLK

    @functools.partial(
        pl.kernel,
        out_type=jax.ShapeDtypeStruct((n_slices * NPAD, BLK), jnp.float32),
        mesh=_MESH,
        scratch_types=[
            pltpu.VMEM_SHARED((NPAD, BLK), jnp.float32),
            pltpu.VMEM((2, CH, IBLK), jnp.int32),
            pltpu.VMEM((2, CH, IBLK), jnp.int32),
            pltpu.VMEM((2, IBLK, BLK), jnp.float32),
            pltpu.SemaphoreType.DMA,
            pltpu.SemaphoreType.DMA,
        ],
    )
    def prop(y_hbm, src_hbm, dst_hbm, agg_hbm, acc, src_c, dst_c, buf, gsem, isem):
        c = lax.axis_index("c")
        w = lax.axis_index("s")
        r0 = w * RPT

        for p in range(passes):
            sl = c * passes + p
            srow0 = sl * nbr + w * NBLK
            drow0 = w * NBLK
            pltpu.sync_copy(
                y_hbm.at[pl.ds(sl * NPAD + r0, RPT)], acc.at[pl.ds(r0, RPT)]
            )
            pltpu.make_async_copy(
                src_hbm.at[pl.ds(srow0, CH)], src_c.at[0], isem
            ).start()
            pltpu.make_async_copy(
                dst_hbm.at[pl.ds(drow0, CH)], dst_c.at[0], isem
            ).start()
            plsc.subcore_barrier()

            @pl.loop(0, NCH, step=2)
            def _(kk):
                for cb in range(2):
                    k = kk + cb
                    pltpu.make_async_copy(
                        src_hbm.at[pl.ds(0, CH)], src_c.at[cb], isem
                    ).wait()
                    pltpu.make_async_copy(
                        src_hbm.at[pl.ds(0, CH)], dst_c.at[cb], isem
                    ).wait()

                    @pl.when(k < NCH - 1)
                    def _():
                        pltpu.make_async_copy(
                            src_hbm.at[pl.ds(srow0 + (k + 1) * CH, CH)],
                            src_c.at[1 - cb], isem,
                        ).start()
                        pltpu.make_async_copy(
                            dst_hbm.at[pl.ds(drow0 + (k + 1) * CH, CH)],
                            dst_c.at[1 - cb], isem,
                        ).start()

                    pltpu.make_async_copy(
                        y_hbm.at[src_c.at[cb, 0]], buf.at[0], gsem
                    ).start()
                    for j2 in range(CH):
                        bb = j2 % 2
                        pltpu.make_async_copy(
                            y_hbm.at[pl.ds(0, IBLK)], buf.at[bb], gsem
                        ).wait()
                        if j2 < CH - 1:
                            pltpu.make_async_copy(
                                y_hbm.at[src_c.at[cb, j2 + 1]],
                                buf.at[1 - bb], gsem,
                            ).start()
                        pltpu.sync_copy(
                            buf.at[bb], acc.at[dst_c.at[cb, j2]], add=True
                        )

            plsc.subcore_barrier()
            pltpu.sync_copy(
                acc.at[pl.ds(r0, RPT)], agg_hbm.at[pl.ds(sl * NPAD + r0, RPT)]
            )
            if p < passes - 1:
                plsc.subcore_barrier()

    return prop(y_flat, src_sh, dst2d)


def _dinv_of(deg_ref):
    return lax.rsqrt(deg_ref[0][:, :1] + deg_ref[1][:, :1] + 1.0)


def _mm1_body(x_ref, w_ref, deg_ref, out_ref):
    d = _dinv_of(deg_ref)
    y = jnp.dot(x_ref[...] * d, w_ref[...], precision=lax.Precision.DEFAULT,
                preferred_element_type=jnp.float32)
    for s in range(HID_DIM // BLK):
        out_ref[s] = y[:, s * BLK:(s + 1) * BLK]


def _mm2_body(agg_ref, deg_ref, b_ref, w_ref, out_ref):
    d = _dinv_of(deg_ref)
    a = jnp.concatenate([agg_ref[s] for s in range(HID_DIM // BLK)], axis=1)
    h = jnp.maximum(a * d + b_ref[...], 0.0)
    y = jnp.dot(h * d, w_ref[...], precision=lax.Precision.DEFAULT,
                preferred_element_type=jnp.float32)
    for s in range(OUT_DIM // BLK):
        out_ref[s] = y[:, s * BLK:(s + 1) * BLK]


def _fin_body(agg_ref, deg_ref, b_ref, out_ref):
    d = _dinv_of(deg_ref)
    a = jnp.concatenate([agg_ref[s] for s in range(OUT_DIM // BLK)], axis=1)
    out_ref[...] = a * d + b_ref[...]


_RB = 256
_GRID = (NPAD // _RB,)


def _deg_spec():
    return pl.BlockSpec((NCORE, _RB, BLK), lambda i: (0, i, 0))


def kernel(x, edge_index, W1, b1, W2, b2):
    src = edge_index[0].astype(jnp.int32)
    dst = edge_index[1].astype(jnp.int32)
    pad_e = EPAD - E
    dummy_src = jnp.arange(pad_e, dtype=jnp.int32) % N
    src2d = jnp.concatenate([src, dummy_src]).reshape(EPAD // IBLK, IBLK)
    dummy_dst = N + (jnp.arange(pad_e, dtype=jnp.int32) % (NPAD - N))
    dst2d = jnp.concatenate([dst, dummy_dst]).reshape(EPAD // IBLK, IBLK)
    xp = jnp.concatenate([x, jnp.zeros((NPAD - N, IN_DIM), x.dtype)])
    zeros_col = jnp.zeros((NPAD, BLK), jnp.float32)
    ones_col = jnp.ones((IBLK, BLK), jnp.float32)

    shifts = (jnp.arange(HID_DIM // BLK, dtype=jnp.int32) * NPAD)[:, None, None]
    src_sh = (src2d[None] + shifts).reshape((HID_DIM // BLK) * (EPAD // IBLK), IBLK)

    deg = _sc_hist(dst2d, zeros_col, ones_col)

    y1 = pl.pallas_call(
        _mm1_body,
        grid=_GRID,
        in_specs=[
            pl.BlockSpec((_RB, IN_DIM), lambda i: (i, 0)),
            pl.BlockSpec((IN_DIM, HID_DIM), lambda i: (0, 0)),
            _deg_spec(),
        ],
        out_specs=pl.BlockSpec((HID_DIM // BLK, _RB, BLK), lambda i: (0, i, 0)),
        out_shape=jax.ShapeDtypeStruct((HID_DIM // BLK, NPAD, BLK), jnp.float32),
    )(xp, W1, deg)

    agg1 = _sc_prop(
        y1.reshape((HID_DIM // BLK) * NPAD, BLK), src_sh, dst2d, HID_DIM // BLK
    ).reshape(HID_DIM // BLK, NPAD, BLK)

    y2 = pl.pallas_call(
        _mm2_body,
        grid=_GRID,
        in_specs=[
            pl.BlockSpec((HID_DIM // BLK, _RB, BLK), lambda i: (0, i, 0)),
            _deg_spec(),
            pl.BlockSpec((1, HID_DIM), lambda i: (0, 0)),
            pl.BlockSpec((HID_DIM, OUT_DIM), lambda i: (0, 0)),
        ],
        out_specs=pl.BlockSpec((OUT_DIM // BLK, _RB, BLK), lambda i: (0, i, 0)),
        out_shape=jax.ShapeDtypeStruct((OUT_DIM // BLK, NPAD, BLK), jnp.float32),
    )(agg1, deg, b1.reshape(1, HID_DIM), W2)

    agg2 = _sc_prop(
        y2.reshape((OUT_DIM // BLK) * NPAD, BLK), src_sh, dst2d, OUT_DIM // BLK
    ).reshape(OUT_DIM // BLK, NPAD, BLK)

    out = pl.pallas_call(
        _fin_body,
        grid=_GRID,
        in_specs=[
            pl.BlockSpec((OUT_DIM // BLK, _RB, BLK), lambda i: (0, i, 0)),
            _deg_spec(),
            pl.BlockSpec((1, OUT_DIM), lambda i: (0, 0)),
        ],
        out_specs=pl.BlockSpec((_RB, OUT_DIM), lambda i: (i, 0)),
        out_shape=jax.ShapeDtypeStruct((N, OUT_DIM), jnp.float32),
    )(agg2, deg, b2.reshape(1, OUT_DIM))

    return out
